# baseline (speedup 1.0000x reference)
import functools

import jax
import jax.numpy as jnp
from jax import lax
from jax.experimental import pallas as pl
from jax.experimental.pallas import tpu as pltpu
from jax.experimental.pallas import tpu_sc as plsc

N_NODES = 10000
N_EDGES = 320000
VOCAB = 128
HIDDEN = 64
N_CLASSES = 20
N_DOCS = 64

NC = 2
NS = 16
NTILES = NC * NS
CH = 128
EPT = 10240
NCH = EPT // CH
E_PAD = NTILES * EPT
CHUNKS = E_PAD // CH
CHUNKS_ALLOC = CHUNKS + 128
NCH0 = 160
NCH1 = 0
C0_CHUNKS = NS * NCH0
PADN = 10112
RPT = PADN // NS

BLK = 2000
NBLK = N_NODES // BLK

_mesh = plsc.VectorSubcoreMesh(
    core_axis_name="c", subcore_axis_name="s", num_cores=NC, num_subcores=NS)
_sc_params = pltpu.CompilerParams(use_tc_tiling_on_sc=False)
_sc_params_nl = pltpu.CompilerParams(use_tc_tiling_on_sc=False,
                                     needs_layout_passes=False)


DEGR = 640
DEG_RPT = DEGR // NS


@functools.partial(
    pl.kernel,
    out_type=jax.ShapeDtypeStruct((NC, DEGR, 16), jnp.float32),
    mesh=_mesh,
    compiler_params=_sc_params_nl,
    scratch_types=[
        pltpu.VMEM((NCH, CH), jnp.int32),
        pltpu.VMEM((5, CH), jnp.int32),
        pltpu.VMEM((DEGR, 16), jnp.float32),
        pltpu.VMEM_SHARED((DEGR, 16), jnp.float32),
    ],
)
def _sc_degree(dsts_hbm, zeros_hbm, ids_hbm, out_hbm, dst_v, ids_v, deg_l, acc):
    c = lax.axis_index("c")
    s = lax.axis_index("s")
    wid = c * NS + s
    pltpu.sync_copy(dsts_hbm.at[pl.ds(wid * NCH, NCH)], dst_v)
    pltpu.sync_copy(ids_hbm, ids_v)
    pltpu.sync_copy(zeros_hbm.at[pl.ds(s * DEG_RPT, DEG_RPT)],
                    acc.at[pl.ds(s * DEG_RPT, DEG_RPT)])
    pltpu.sync_copy(zeros_hbm, deg_l)
    ones = jnp.ones((16,), jnp.float32)

    def step(i, carry):
        j = i // 8
        col = (i % 8) * 16
        idx = dst_v[j, pl.ds(col, 16)]
        plsc.addupdate_scatter(deg_l, [idx >> 4, idx & 15], ones)
        return carry

    lax.fori_loop(0, EPT // 16, step, 0)
    plsc.subcore_barrier()
    for k in range(DEGR // CH):
        pltpu.sync_copy(deg_l.at[pl.ds(k * CH, CH)], acc.at[ids_v.at[k]],
                        add=True)
    plsc.subcore_barrier()
    pltpu.sync_copy(acc.at[pl.ds(s * DEG_RPT, DEG_RPT)],
                    out_hbm.at[c, pl.ds(s * DEG_RPT, DEG_RPT)])


@functools.partial(
    pl.kernel,
    out_type=jax.ShapeDtypeStruct((NC, PADN, HIDDEN), jnp.float32),
    mesh=_mesh,
    compiler_params=_sc_params,
    scratch_types=[
        pltpu.VMEM((NCH, CH), jnp.int32),
        pltpu.VMEM((NCH, CH), jnp.int32),
        [pltpu.VMEM((CH, HIDDEN), jnp.float32)] * 4,
        pltpu.VMEM_SHARED((PADN, HIDDEN), jnp.float32),
        [pltpu.SemaphoreType.DMA] * 4,
        [pltpu.SemaphoreType.DMA] * 4,
    ],
)
def _sc_propagate(g_hbm, srcs_hbm, dsts_hbm, out_hbm,
                  src_v, dst_v, rows, acc, gsems, ssems):
    c = lax.axis_index("c")
    s = lax.axis_index("s")

    def zrow(i, carry):
        col = (i % 4) * 16
        rows[0][i // 4, pl.ds(col, 16)] = jnp.zeros((16,), jnp.float32)
        return carry

    lax.fori_loop(0, CH * 4, zrow, 0)
    for k in range(5):
        n = CH if k < 4 else RPT - 4 * CH
        pltpu.sync_copy(rows[0].at[pl.ds(0, n)],
                        acc.at[pl.ds(s * RPT + k * CH, n)])
    plsc.subcore_barrier()

    def chunk(i, carry):
        j = i * 4
        g = [pltpu.async_copy(g_hbm.at[src_v.at[j + k]], rows[k], gsems[k])
             for k in range(4)]
        sc = []
        for k in range(4):
            g[k].wait()
            sc.append(pltpu.async_copy(rows[k], acc.at[dst_v.at[j + k]],
                                       ssems[k], add=True))
        for k in range(4):
            sc[k].wait()
        return carry

    def round_(r, carry):
        pltpu.sync_copy(srcs_hbm.at[pl.ds(s * NCH0 + r * NCH, NCH)], src_v)
        pltpu.sync_copy(dsts_hbm.at[pl.ds(s * NCH0 + r * NCH, NCH)], dst_v)
        lax.fori_loop(0, NCH // 4, chunk, 0)
        return carry

    lax.fori_loop(0, jnp.where(c == 0, NCH0 // NCH, 0), round_, 0)
    plsc.subcore_barrier()
    pltpu.sync_copy(acc.at[pl.ds(s * RPT, RPT)],
                    out_hbm.at[c, pl.ds(s * RPT, RPT)])


def _tc1_body(x_ref, w1_ref, d0_ref, d1_ref, dinv_ref, g1_ref):
    deg = d0_ref[...] + d1_ref[...] + 1.0
    dinv = lax.rsqrt(deg)
    dinv_ref[...] = dinv
    h = jnp.dot(x_ref[...], w1_ref[...], preferred_element_type=jnp.float32)
    g1_ref[...] = h * dinv


def _tc2_body(p0_ref, p1_ref, g1_ref, dinv_ref, b1_ref, w2_ref, g2_ref):
    dinv = dinv_ref[...]
    pre = dinv * (p0_ref[...] + p1_ref[...] + g1_ref[...]) + b1_ref[...]
    out1 = jnp.maximum(pre, 0.0)
    g2_ref[...] = jnp.dot(out1, w2_ref[...],
                          preferred_element_type=jnp.float32) * dinv


def _tc3_body(q0_ref, q1_ref, g2_ref, dinv_ref, b2_ref, batch_ref,
              wc_ref, bc_ref, out_ref, acc_ref):
    i = pl.program_id(0)

    @pl.when(i == 0)
    def _():
        acc_ref[...] = jnp.zeros_like(acc_ref)

    dinv = dinv_ref[...]
    pre = dinv * (q0_ref[...] + q1_ref[...] + g2_ref[...]) + b2_ref[...]
    h = jnp.maximum(pre, 0.0)
    hx = jnp.concatenate([h, jnp.ones((BLK, 1), jnp.float32)], axis=1)
    docs = lax.broadcasted_iota(jnp.int32, (N_DOCS, 1), 0).astype(jnp.float32)
    onehot_t = (docs == batch_ref[0]).astype(jnp.float32)
    acc_ref[...] += jnp.dot(onehot_t, hx,
                            preferred_element_type=jnp.float32)

    @pl.when(i == NBLK - 1)
    def _():
        sums = acc_ref[:, :HIDDEN]
        counts = acc_ref[:, HIDDEN:HIDDEN + 1]
        pooled = sums / jnp.maximum(counts, 1.0)
        logits = jnp.dot(pooled, wc_ref[...],
                         preferred_element_type=jnp.float32) + bc_ref[...]
        m = jnp.max(logits, axis=1, keepdims=True)
        lse = jnp.log(jnp.sum(jnp.exp(logits - m), axis=1, keepdims=True)) + m
        out_ref[...] = logits - lse


_tc1 = pl.pallas_call(
    _tc1_body,
    grid=(NBLK,),
    in_specs=[
        pl.BlockSpec((BLK, VOCAB), lambda i: (i, 0)),
        pl.BlockSpec((VOCAB, HIDDEN), lambda i: (0, 0)),
        pl.BlockSpec((BLK, 1), lambda i: (i, 0)),
        pl.BlockSpec((BLK, 1), lambda i: (i, 0)),
    ],
    out_specs=[
        pl.BlockSpec((BLK, 1), lambda i: (i, 0)),
        pl.BlockSpec((BLK, HIDDEN), lambda i: (i, 0)),
    ],
    out_shape=[
        jax.ShapeDtypeStruct((N_NODES, 1), jnp.float32),
        jax.ShapeDtypeStruct((N_NODES, HIDDEN), jnp.float32),
    ],
)

_tc2 = pl.pallas_call(
    _tc2_body,
    grid=(NBLK,),
    in_specs=[
        pl.BlockSpec((BLK, HIDDEN), lambda i: (i, 0)),
        pl.BlockSpec((BLK, HIDDEN), lambda i: (i, 0)),
        pl.BlockSpec((BLK, HIDDEN), lambda i: (i, 0)),
        pl.BlockSpec((BLK, 1), lambda i: (i, 0)),
        pl.BlockSpec((1, HIDDEN), lambda i: (0, 0)),
        pl.BlockSpec((HIDDEN, HIDDEN), lambda i: (0, 0)),
    ],
    out_specs=pl.BlockSpec((BLK, HIDDEN), lambda i: (i, 0)),
    out_shape=jax.ShapeDtypeStruct((N_NODES, HIDDEN), jnp.float32),
)

_tc3 = pl.pallas_call(
    _tc3_body,
    grid=(NBLK,),
    in_specs=[
        pl.BlockSpec((BLK, HIDDEN), lambda i: (i, 0)),
        pl.BlockSpec((BLK, HIDDEN), lambda i: (i, 0)),
        pl.BlockSpec((BLK, HIDDEN), lambda i: (i, 0)),
        pl.BlockSpec((BLK, 1), lambda i: (i, 0)),
        pl.BlockSpec((1, HIDDEN), lambda i: (0, 0)),
        pl.BlockSpec((1, 1, BLK), lambda i: (i, 0, 0)),
        pl.BlockSpec((HIDDEN, N_CLASSES), lambda i: (0, 0)),
        pl.BlockSpec((1, N_CLASSES), lambda i: (0, 0)),
    ],
    out_specs=pl.BlockSpec((N_DOCS, N_CLASSES), lambda i: (0, 0)),
    out_shape=jax.ShapeDtypeStruct((N_DOCS, N_CLASSES), jnp.float32),
    scratch_shapes=[pltpu.VMEM((N_DOCS, HIDDEN + 1), jnp.float32)],
)


def kernel(x, edge_index, batch, W1, b1, W2, b2, Wc, bc):
    src = edge_index[0].astype(jnp.int32)
    dst = edge_index[1].astype(jnp.int32)
    pad = CHUNKS_ALLOC * CH - N_EDGES
    src_p = jnp.concatenate([src, jnp.zeros((pad,), jnp.int32)])
    dst_p = jnp.concatenate([dst, jnp.full((pad,), N_NODES, jnp.int32)])
    src_p = src_p.reshape(CHUNKS_ALLOC, CH)
    dst_p = dst_p.reshape(CHUNKS_ALLOC, CH)

    zerosd = jnp.zeros((DEGR, 16), jnp.float32)
    ids = jnp.arange(DEGR, dtype=jnp.int32).reshape(5, CH)

    degw = _sc_degree(dst_p, zerosd, ids)
    d0 = degw[0].reshape(DEGR * 16)[:N_NODES, None]
    d1 = degw[1].reshape(DEGR * 16)[:N_NODES, None]

    dinv, g1 = _tc1(x, W1, d0, d1)

    P = _sc_propagate(g1, src_p, dst_p)
    g2 = _tc2(P[0, :N_NODES], P[1, :N_NODES], g1, dinv,
              b1.reshape(1, HIDDEN), W2)

    Q = _sc_propagate(g2, src_p, dst_p)
    out = _tc3(Q[0, :N_NODES], Q[1, :N_NODES], g2, dinv,
               b2.reshape(1, HIDDEN),
               batch.astype(jnp.float32).reshape(NBLK, 1, BLK),
               Wc, bc.reshape(1, N_CLASSES))
    return out


# speedup vs baseline: 2.7111x; 2.7111x over previous
import functools

import jax
import jax.numpy as jnp
from jax import lax
from jax.experimental import pallas as pl
from jax.experimental.pallas import tpu as pltpu
from jax.experimental.pallas import tpu_sc as plsc

N_NODES = 10000
N_EDGES = 320000
VOCAB = 128
HIDDEN = 64
N_CLASSES = 20
N_DOCS = 64

NC = 2
NS = 16
NTILES = NC * NS
CH = 128
EPT = 10240
NCH = EPT // CH
E_PAD = NTILES * EPT
CHUNKS = E_PAD // CH
PADN = 10112
RPT = PADN // NS

BLK = 2000
NBLK = N_NODES // BLK

_mesh = plsc.VectorSubcoreMesh(
    core_axis_name="c", subcore_axis_name="s", num_cores=NC, num_subcores=NS)
_sc_params = pltpu.CompilerParams(use_tc_tiling_on_sc=False)
_sc_params_nl = pltpu.CompilerParams(use_tc_tiling_on_sc=False,
                                     needs_layout_passes=False)


DEGR = 640
DEG_RPT = DEGR // NS


@functools.partial(
    pl.kernel,
    out_type=jax.ShapeDtypeStruct((NC, DEGR, 16), jnp.float32),
    mesh=_mesh,
    compiler_params=_sc_params_nl,
    scratch_types=[
        pltpu.VMEM((NCH, CH), jnp.int32),
        pltpu.VMEM((5, CH), jnp.int32),
        pltpu.VMEM((DEGR, 16), jnp.float32),
        pltpu.VMEM_SHARED((DEGR, 16), jnp.float32),
    ],
)
def _sc_degree(dsts_hbm, zeros_hbm, ids_hbm, out_hbm, dst_v, ids_v, deg_l, acc):
    c = lax.axis_index("c")
    s = lax.axis_index("s")
    wid = c * NS + s
    pltpu.sync_copy(dsts_hbm.at[pl.ds(wid * NCH, NCH)], dst_v)
    pltpu.sync_copy(ids_hbm, ids_v)
    pltpu.sync_copy(zeros_hbm.at[pl.ds(s * DEG_RPT, DEG_RPT)],
                    acc.at[pl.ds(s * DEG_RPT, DEG_RPT)])
    pltpu.sync_copy(zeros_hbm, deg_l)
    ones = jnp.ones((16,), jnp.float32)

    def step(i, carry):
        j = i // 8
        col = (i % 8) * 16
        idx = dst_v[j, pl.ds(col, 16)]
        plsc.addupdate_scatter(deg_l, [idx >> 4, idx & 15], ones)
        return carry

    lax.fori_loop(0, EPT // 16, step, 0)
    plsc.subcore_barrier()
    for k in range(DEGR // CH):
        pltpu.sync_copy(deg_l.at[pl.ds(k * CH, CH)], acc.at[ids_v.at[k]],
                        add=True)
    plsc.subcore_barrier()
    pltpu.sync_copy(acc.at[pl.ds(s * DEG_RPT, DEG_RPT)],
                    out_hbm.at[c, pl.ds(s * DEG_RPT, DEG_RPT)])


@functools.partial(
    pl.kernel,
    out_type=jax.ShapeDtypeStruct((NC, PADN, HIDDEN), jnp.float32),
    mesh=_mesh,
    compiler_params=_sc_params,
    scratch_types=[
        pltpu.VMEM((NCH, CH), jnp.int32),
        pltpu.VMEM((NCH, CH), jnp.int32),
        [pltpu.VMEM((CH, HIDDEN), jnp.float32)] * 4,
        pltpu.VMEM_SHARED((PADN, HIDDEN), jnp.float32),
        [pltpu.SemaphoreType.DMA] * 4,
        [pltpu.SemaphoreType.DMA] * 4,
    ],
)
def _sc_propagate(g_hbm, srcs_hbm, dsts_hbm, out_hbm,
                  src_v, dst_v, rows, acc, gsems, ssems):
    c = lax.axis_index("c")
    s = lax.axis_index("s")

    def zrow(i, carry):
        col = (i % 4) * 16
        rows[0][i // 4, pl.ds(col, 16)] = jnp.zeros((16,), jnp.float32)
        return carry

    lax.fori_loop(0, CH * 4, zrow, 0)
    for k in range(5):
        n = CH if k < 4 else RPT - 4 * CH
        pltpu.sync_copy(rows[0].at[pl.ds(0, n)],
                        acc.at[pl.ds(s * RPT + k * CH, n)])
    plsc.subcore_barrier()

    def chunk(i, carry):
        j = i * 4
        g = [pltpu.async_copy(g_hbm.at[src_v.at[j + k]], rows[k], gsems[k])
             for k in range(4)]
        sc = []
        for k in range(4):
            g[k].wait()
            sc.append(pltpu.async_copy(rows[k], acc.at[dst_v.at[j + k]],
                                       ssems[k], add=True))
        for k in range(4):
            sc[k].wait()
        return carry

    wid = c * NS + s
    pltpu.sync_copy(srcs_hbm.at[pl.ds(wid * NCH, NCH)], src_v)
    pltpu.sync_copy(dsts_hbm.at[pl.ds(wid * NCH, NCH)], dst_v)
    lax.fori_loop(0, NCH // 4, chunk, 0)
    plsc.subcore_barrier()
    pltpu.sync_copy(acc.at[pl.ds(s * RPT, RPT)],
                    out_hbm.at[c, pl.ds(s * RPT, RPT)])


def _tc1_body(x_ref, w1_ref, d0_ref, d1_ref, dinv_ref, g1_ref):
    deg = d0_ref[...] + d1_ref[...] + 1.0
    dinv = lax.rsqrt(deg)
    dinv_ref[...] = dinv
    h = jnp.dot(x_ref[...], w1_ref[...], preferred_element_type=jnp.float32)
    g1_ref[...] = h * dinv


def _tc2_body(p0_ref, p1_ref, g1_ref, dinv_ref, b1_ref, w2_ref, g2_ref):
    dinv = dinv_ref[...]
    pre = dinv * (p0_ref[...] + p1_ref[...] + g1_ref[...]) + b1_ref[...]
    out1 = jnp.maximum(pre, 0.0)
    g2_ref[...] = jnp.dot(out1, w2_ref[...],
                          preferred_element_type=jnp.float32) * dinv


def _tc3_body(q0_ref, q1_ref, g2_ref, dinv_ref, b2_ref, batch_ref,
              wc_ref, bc_ref, out_ref, acc_ref):
    i = pl.program_id(0)

    @pl.when(i == 0)
    def _():
        acc_ref[...] = jnp.zeros_like(acc_ref)

    dinv = dinv_ref[...]
    pre = dinv * (q0_ref[...] + q1_ref[...] + g2_ref[...]) + b2_ref[...]
    h = jnp.maximum(pre, 0.0)
    hx = jnp.concatenate([h, jnp.ones((BLK, 1), jnp.float32)], axis=1)
    docs = lax.broadcasted_iota(jnp.int32, (N_DOCS, 1), 0).astype(jnp.float32)
    onehot_t = (docs == batch_ref[0]).astype(jnp.float32)
    acc_ref[...] += jnp.dot(onehot_t, hx,
                            preferred_element_type=jnp.float32)

    @pl.when(i == NBLK - 1)
    def _():
        sums = acc_ref[:, :HIDDEN]
        counts = acc_ref[:, HIDDEN:HIDDEN + 1]
        pooled = sums / jnp.maximum(counts, 1.0)
        logits = jnp.dot(pooled, wc_ref[...],
                         preferred_element_type=jnp.float32) + bc_ref[...]
        m = jnp.max(logits, axis=1, keepdims=True)
        lse = jnp.log(jnp.sum(jnp.exp(logits - m), axis=1, keepdims=True)) + m
        out_ref[...] = logits - lse


_tc1 = pl.pallas_call(
    _tc1_body,
    grid=(NBLK,),
    in_specs=[
        pl.BlockSpec((BLK, VOCAB), lambda i: (i, 0)),
        pl.BlockSpec((VOCAB, HIDDEN), lambda i: (0, 0)),
        pl.BlockSpec((BLK, 1), lambda i: (i, 0)),
        pl.BlockSpec((BLK, 1), lambda i: (i, 0)),
    ],
    out_specs=[
        pl.BlockSpec((BLK, 1), lambda i: (i, 0)),
        pl.BlockSpec((BLK, HIDDEN), lambda i: (i, 0)),
    ],
    out_shape=[
        jax.ShapeDtypeStruct((N_NODES, 1), jnp.float32),
        jax.ShapeDtypeStruct((N_NODES, HIDDEN), jnp.float32),
    ],
)

_tc2 = pl.pallas_call(
    _tc2_body,
    grid=(NBLK,),
    in_specs=[
        pl.BlockSpec((BLK, HIDDEN), lambda i: (i, 0)),
        pl.BlockSpec((BLK, HIDDEN), lambda i: (i, 0)),
        pl.BlockSpec((BLK, HIDDEN), lambda i: (i, 0)),
        pl.BlockSpec((BLK, 1), lambda i: (i, 0)),
        pl.BlockSpec((1, HIDDEN), lambda i: (0, 0)),
        pl.BlockSpec((HIDDEN, HIDDEN), lambda i: (0, 0)),
    ],
    out_specs=pl.BlockSpec((BLK, HIDDEN), lambda i: (i, 0)),
    out_shape=jax.ShapeDtypeStruct((N_NODES, HIDDEN), jnp.float32),
)

_tc3 = pl.pallas_call(
    _tc3_body,
    grid=(NBLK,),
    in_specs=[
        pl.BlockSpec((BLK, HIDDEN), lambda i: (i, 0)),
        pl.BlockSpec((BLK, HIDDEN), lambda i: (i, 0)),
        pl.BlockSpec((BLK, HIDDEN), lambda i: (i, 0)),
        pl.BlockSpec((BLK, 1), lambda i: (i, 0)),
        pl.BlockSpec((1, HIDDEN), lambda i: (0, 0)),
        pl.BlockSpec((1, 1, BLK), lambda i: (i, 0, 0)),
        pl.BlockSpec((HIDDEN, N_CLASSES), lambda i: (0, 0)),
        pl.BlockSpec((1, N_CLASSES), lambda i: (0, 0)),
    ],
    out_specs=pl.BlockSpec((N_DOCS, N_CLASSES), lambda i: (0, 0)),
    out_shape=jax.ShapeDtypeStruct((N_DOCS, N_CLASSES), jnp.float32),
    scratch_shapes=[pltpu.VMEM((N_DOCS, HIDDEN + 1), jnp.float32)],
)


def kernel(x, edge_index, batch, W1, b1, W2, b2, Wc, bc):
    src = edge_index[0].astype(jnp.int32)
    dst = edge_index[1].astype(jnp.int32)
    pad = E_PAD - N_EDGES
    pad_idx = jnp.arange(pad, dtype=jnp.int32)
    src_p = jnp.concatenate([src, pad_idx % N_NODES])
    dst_p = jnp.concatenate([dst, N_NODES + pad_idx % (PADN - N_NODES)])
    src_p = src_p.reshape(CHUNKS, CH)
    dst_p = dst_p.reshape(CHUNKS, CH)

    zerosd = jnp.zeros((DEGR, 16), jnp.float32)
    ids = jnp.arange(DEGR, dtype=jnp.int32).reshape(5, CH)

    degw = _sc_degree(dst_p, zerosd, ids)
    d0 = degw[0].reshape(DEGR * 16)[:N_NODES, None]
    d1 = degw[1].reshape(DEGR * 16)[:N_NODES, None]

    dinv, g1 = _tc1(x, W1, d0, d1)

    P = _sc_propagate(g1, src_p, dst_p)
    g2 = _tc2(P[0, :N_NODES], P[1, :N_NODES], g1, dinv,
              b1.reshape(1, HIDDEN), W2)

    Q = _sc_propagate(g2, src_p, dst_p)
    out = _tc3(Q[0, :N_NODES], Q[1, :N_NODES], g2, dinv,
               b2.reshape(1, HIDDEN),
               batch.astype(jnp.float32).reshape(NBLK, 1, BLK),
               Wc, bc.reshape(1, N_CLASSES))
    return out


# speedup vs baseline: 2.7733x; 1.0230x over previous
import functools

import jax
import jax.numpy as jnp
from jax import lax
from jax.experimental import pallas as pl
from jax.experimental.pallas import tpu as pltpu
from jax.experimental.pallas import tpu_sc as plsc

N_NODES = 10000
N_EDGES = 320000
VOCAB = 128
HIDDEN = 64
N_CLASSES = 20
N_DOCS = 64

NC = 2
NS = 16
NTILES = NC * NS
CH = 128
EPT = 10240
NCH = EPT // CH
E_PAD = NTILES * EPT
CHUNKS = E_PAD // CH
PADN = 10112
RPT = PADN // NS

BLK = 2000
NBLK = N_NODES // BLK

_mesh = plsc.VectorSubcoreMesh(
    core_axis_name="c", subcore_axis_name="s", num_cores=NC, num_subcores=NS)
_sc_params = pltpu.CompilerParams(use_tc_tiling_on_sc=False)
_sc_params_nl = pltpu.CompilerParams(use_tc_tiling_on_sc=False,
                                     needs_layout_passes=False)


DEGR = 640
DEG_RPT = DEGR // NS


@functools.partial(
    pl.kernel,
    out_type=jax.ShapeDtypeStruct((NC, DEGR, 16), jnp.float32),
    mesh=_mesh,
    compiler_params=_sc_params_nl,
    scratch_types=[
        pltpu.VMEM((NCH, CH), jnp.int32),
        pltpu.VMEM((5, CH), jnp.int32),
        pltpu.VMEM((DEGR, 16), jnp.float32),
        pltpu.VMEM_SHARED((DEGR, 16), jnp.float32),
    ],
)
def _sc_degree(dsts_hbm, zeros_hbm, ids_hbm, out_hbm, dst_v, ids_v, deg_l, acc):
    c = lax.axis_index("c")
    s = lax.axis_index("s")
    wid = c * NS + s
    pltpu.sync_copy(dsts_hbm.at[pl.ds(wid * NCH, NCH)], dst_v)
    pltpu.sync_copy(ids_hbm, ids_v)
    pltpu.sync_copy(zeros_hbm.at[pl.ds(s * DEG_RPT, DEG_RPT)],
                    acc.at[pl.ds(s * DEG_RPT, DEG_RPT)])
    pltpu.sync_copy(zeros_hbm, deg_l)
    ones = jnp.ones((16,), jnp.float32)

    def step(i, carry):
        j = i // 8
        col = (i % 8) * 16
        idx = dst_v[j, pl.ds(col, 16)]
        plsc.addupdate_scatter(deg_l, [idx >> 4, idx & 15], ones)
        return carry

    lax.fori_loop(0, EPT // 16, step, 0)
    plsc.subcore_barrier()
    for k in range(DEGR // CH):
        pltpu.sync_copy(deg_l.at[pl.ds(k * CH, CH)], acc.at[ids_v.at[k]],
                        add=True)
    plsc.subcore_barrier()
    pltpu.sync_copy(acc.at[pl.ds(s * DEG_RPT, DEG_RPT)],
                    out_hbm.at[c, pl.ds(s * DEG_RPT, DEG_RPT)])


@functools.partial(
    pl.kernel,
    out_type=jax.ShapeDtypeStruct((NC, PADN, HIDDEN), jnp.float32),
    mesh=_mesh,
    compiler_params=_sc_params,
    scratch_types=[
        pltpu.VMEM((NCH, CH), jnp.int32),
        pltpu.VMEM((NCH, CH), jnp.int32),
        [pltpu.VMEM((CH, HIDDEN), jnp.float32)] * 5,
        pltpu.VMEM_SHARED((PADN, HIDDEN), jnp.float32),
        [pltpu.SemaphoreType.DMA] * 5,
        [pltpu.SemaphoreType.DMA] * 5,
    ],
)
def _sc_propagate(g_hbm, srcs_hbm, dsts_hbm, out_hbm,
                  src_v, dst_v, rows, acc, gsems, ssems):
    c = lax.axis_index("c")
    s = lax.axis_index("s")

    def zrow(i, carry):
        col = (i % 4) * 16
        rows[0][i // 4, pl.ds(col, 16)] = jnp.zeros((16,), jnp.float32)
        return carry

    lax.fori_loop(0, CH * 4, zrow, 0)
    for k in range(5):
        n = CH if k < 4 else RPT - 4 * CH
        pltpu.sync_copy(rows[0].at[pl.ds(0, n)],
                        acc.at[pl.ds(s * RPT + k * CH, n)])
    plsc.subcore_barrier()

    def chunk(i, carry):
        j = i * 5
        g = [pltpu.async_copy(g_hbm.at[src_v.at[j + k]], rows[k], gsems[k])
             for k in range(5)]
        sc = []
        for k in range(5):
            g[k].wait()
            sc.append(pltpu.async_copy(rows[k], acc.at[dst_v.at[j + k]],
                                       ssems[k], add=True))
        for k in range(5):
            sc[k].wait()
        return carry

    wid = c * NS + s
    pltpu.sync_copy(srcs_hbm.at[pl.ds(wid * NCH, NCH)], src_v)
    pltpu.sync_copy(dsts_hbm.at[pl.ds(wid * NCH, NCH)], dst_v)
    lax.fori_loop(0, NCH // 5, chunk, 0)
    plsc.subcore_barrier()
    pltpu.sync_copy(acc.at[pl.ds(s * RPT, RPT)],
                    out_hbm.at[c, pl.ds(s * RPT, RPT)])


def _tc1_body(x_ref, w1_ref, d0_ref, d1_ref, dinv_ref, g1_ref):
    deg = d0_ref[...] + d1_ref[...] + 1.0
    dinv = lax.rsqrt(deg)
    dinv_ref[...] = dinv
    h = jnp.dot(x_ref[...], w1_ref[...], preferred_element_type=jnp.float32)
    g1_ref[...] = h * dinv


def _tc2_body(p0_ref, p1_ref, g1_ref, dinv_ref, b1_ref, w2_ref, g2_ref):
    dinv = dinv_ref[...]
    pre = dinv * (p0_ref[...] + p1_ref[...] + g1_ref[...]) + b1_ref[...]
    out1 = jnp.maximum(pre, 0.0)
    g2_ref[...] = jnp.dot(out1, w2_ref[...],
                          preferred_element_type=jnp.float32) * dinv


def _tc3_body(q0_ref, q1_ref, g2_ref, dinv_ref, b2_ref, batch_ref,
              wc_ref, bc_ref, out_ref, acc_ref):
    i = pl.program_id(0)

    @pl.when(i == 0)
    def _():
        acc_ref[...] = jnp.zeros_like(acc_ref)

    dinv = dinv_ref[...]
    pre = dinv * (q0_ref[...] + q1_ref[...] + g2_ref[...]) + b2_ref[...]
    h = jnp.maximum(pre, 0.0)
    hx = jnp.concatenate([h, jnp.ones((BLK, 1), jnp.float32)], axis=1)
    docs = lax.broadcasted_iota(jnp.int32, (N_DOCS, 1), 0).astype(jnp.float32)
    onehot_t = (docs == batch_ref[0]).astype(jnp.float32)
    acc_ref[...] += jnp.dot(onehot_t, hx,
                            preferred_element_type=jnp.float32)

    @pl.when(i == NBLK - 1)
    def _():
        sums = acc_ref[:, :HIDDEN]
        counts = acc_ref[:, HIDDEN:HIDDEN + 1]
        pooled = sums / jnp.maximum(counts, 1.0)
        logits = jnp.dot(pooled, wc_ref[...],
                         preferred_element_type=jnp.float32) + bc_ref[...]
        m = jnp.max(logits, axis=1, keepdims=True)
        lse = jnp.log(jnp.sum(jnp.exp(logits - m), axis=1, keepdims=True)) + m
        out_ref[...] = logits - lse


_tc1 = pl.pallas_call(
    _tc1_body,
    grid=(NBLK,),
    in_specs=[
        pl.BlockSpec((BLK, VOCAB), lambda i: (i, 0)),
        pl.BlockSpec((VOCAB, HIDDEN), lambda i: (0, 0)),
        pl.BlockSpec((BLK, 1), lambda i: (i, 0)),
        pl.BlockSpec((BLK, 1), lambda i: (i, 0)),
    ],
    out_specs=[
        pl.BlockSpec((BLK, 1), lambda i: (i, 0)),
        pl.BlockSpec((BLK, HIDDEN), lambda i: (i, 0)),
    ],
    out_shape=[
        jax.ShapeDtypeStruct((N_NODES, 1), jnp.float32),
        jax.ShapeDtypeStruct((N_NODES, HIDDEN), jnp.float32),
    ],
)

_tc2 = pl.pallas_call(
    _tc2_body,
    grid=(NBLK,),
    in_specs=[
        pl.BlockSpec((BLK, HIDDEN), lambda i: (i, 0)),
        pl.BlockSpec((BLK, HIDDEN), lambda i: (i, 0)),
        pl.BlockSpec((BLK, HIDDEN), lambda i: (i, 0)),
        pl.BlockSpec((BLK, 1), lambda i: (i, 0)),
        pl.BlockSpec((1, HIDDEN), lambda i: (0, 0)),
        pl.BlockSpec((HIDDEN, HIDDEN), lambda i: (0, 0)),
    ],
    out_specs=pl.BlockSpec((BLK, HIDDEN), lambda i: (i, 0)),
    out_shape=jax.ShapeDtypeStruct((N_NODES, HIDDEN), jnp.float32),
)

_tc3 = pl.pallas_call(
    _tc3_body,
    grid=(NBLK,),
    in_specs=[
        pl.BlockSpec((BLK, HIDDEN), lambda i: (i, 0)),
        pl.BlockSpec((BLK, HIDDEN), lambda i: (i, 0)),
        pl.BlockSpec((BLK, HIDDEN), lambda i: (i, 0)),
        pl.BlockSpec((BLK, 1), lambda i: (i, 0)),
        pl.BlockSpec((1, HIDDEN), lambda i: (0, 0)),
        pl.BlockSpec((1, 1, BLK), lambda i: (i, 0, 0)),
        pl.BlockSpec((HIDDEN, N_CLASSES), lambda i: (0, 0)),
        pl.BlockSpec((1, N_CLASSES), lambda i: (0, 0)),
    ],
    out_specs=pl.BlockSpec((N_DOCS, N_CLASSES), lambda i: (0, 0)),
    out_shape=jax.ShapeDtypeStruct((N_DOCS, N_CLASSES), jnp.float32),
    scratch_shapes=[pltpu.VMEM((N_DOCS, HIDDEN + 1), jnp.float32)],
)


def kernel(x, edge_index, batch, W1, b1, W2, b2, Wc, bc):
    src = edge_index[0].astype(jnp.int32)
    dst = edge_index[1].astype(jnp.int32)
    pad = E_PAD - N_EDGES
    pad_idx = jnp.arange(pad, dtype=jnp.int32)
    src_p = jnp.concatenate([src, pad_idx % N_NODES])
    dst_p = jnp.concatenate([dst, N_NODES + pad_idx % (PADN - N_NODES)])
    src_p = src_p.reshape(CHUNKS, CH)
    dst_p = dst_p.reshape(CHUNKS, CH)

    zerosd = jnp.zeros((DEGR, 16), jnp.float32)
    ids = jnp.arange(DEGR, dtype=jnp.int32).reshape(5, CH)

    degw = _sc_degree(dst_p, zerosd, ids)
    d0 = degw[0].reshape(DEGR * 16)[:N_NODES, None]
    d1 = degw[1].reshape(DEGR * 16)[:N_NODES, None]

    dinv, g1 = _tc1(x, W1, d0, d1)

    P = _sc_propagate(g1, src_p, dst_p)
    g2 = _tc2(P[0, :N_NODES], P[1, :N_NODES], g1, dinv,
              b1.reshape(1, HIDDEN), W2)

    Q = _sc_propagate(g2, src_p, dst_p)
    out = _tc3(Q[0, :N_NODES], Q[1, :N_NODES], g2, dinv,
               b2.reshape(1, HIDDEN),
               batch.astype(jnp.float32).reshape(NBLK, 1, BLK),
               Wc, bc.reshape(1, N_CLASSES))
    return out


# speedup vs baseline: 2.7895x; 1.0058x over previous
import functools

import jax
import jax.numpy as jnp
from jax import lax
from jax.experimental import pallas as pl
from jax.experimental.pallas import tpu as pltpu
from jax.experimental.pallas import tpu_sc as plsc

N_NODES = 10000
N_EDGES = 320000
VOCAB = 128
HIDDEN = 64
N_CLASSES = 20
N_DOCS = 64

NC = 2
NS = 16
NTILES = NC * NS
CH = 128
EPT = 10240
NCH = EPT // CH
CHUNKS = N_EDGES // CH
TAIL_NCH = CHUNKS - (NTILES - 1) * NCH
PADN = 10112
RPT = PADN // NS

BLK = 2000
NBLK = N_NODES // BLK

_mesh = plsc.VectorSubcoreMesh(
    core_axis_name="c", subcore_axis_name="s", num_cores=NC, num_subcores=NS)
_sc_params = pltpu.CompilerParams(use_tc_tiling_on_sc=False)
_sc_params_nl = pltpu.CompilerParams(use_tc_tiling_on_sc=False,
                                     needs_layout_passes=False)


DEGR = 640
DEG_RPT = DEGR // NS


@functools.partial(
    pl.kernel,
    out_type=jax.ShapeDtypeStruct((NC, DEGR, 16), jnp.float32),
    mesh=_mesh,
    compiler_params=_sc_params_nl,
    scratch_types=[
        pltpu.VMEM((NCH, CH), jnp.int32),
        pltpu.VMEM((5, CH), jnp.int32),
        pltpu.VMEM((DEGR, 16), jnp.float32),
        pltpu.VMEM_SHARED((DEGR, 16), jnp.float32),
    ],
)
def _sc_degree(dsts_hbm, zeros_hbm, ids_hbm, out_hbm, dst_v, ids_v, deg_l, acc):
    c = lax.axis_index("c")
    s = lax.axis_index("s")
    wid = c * NS + s

    @pl.when(wid < NTILES - 1)
    def _():
        pltpu.sync_copy(dsts_hbm.at[pl.ds(wid * NCH, NCH)], dst_v)

    @pl.when(wid == NTILES - 1)
    def _():
        pltpu.sync_copy(dsts_hbm.at[pl.ds((NTILES - 1) * NCH, TAIL_NCH)],
                        dst_v.at[pl.ds(0, TAIL_NCH)])

    pltpu.sync_copy(ids_hbm, ids_v)
    pltpu.sync_copy(zeros_hbm.at[pl.ds(s * DEG_RPT, DEG_RPT)],
                    acc.at[pl.ds(s * DEG_RPT, DEG_RPT)])
    pltpu.sync_copy(zeros_hbm, deg_l)
    ones = jnp.ones((16,), jnp.float32)

    def step(i, carry):
        j = i // 8
        col = (i % 8) * 16
        idx = dst_v[j, pl.ds(col, 16)]
        plsc.addupdate_scatter(deg_l, [idx >> 4, idx & 15], ones)
        return carry

    lax.fori_loop(0, jnp.where(wid == NTILES - 1, TAIL_NCH * 8, NCH * 8),
                  step, 0)
    plsc.subcore_barrier()
    for k in range(DEGR // CH):
        pltpu.sync_copy(deg_l.at[pl.ds(k * CH, CH)], acc.at[ids_v.at[k]],
                        add=True)
    plsc.subcore_barrier()
    pltpu.sync_copy(acc.at[pl.ds(s * DEG_RPT, DEG_RPT)],
                    out_hbm.at[c, pl.ds(s * DEG_RPT, DEG_RPT)])


@functools.partial(
    pl.kernel,
    out_type=jax.ShapeDtypeStruct((NC, PADN, HIDDEN), jnp.float32),
    mesh=_mesh,
    compiler_params=_sc_params,
    scratch_types=[
        pltpu.VMEM((NCH, CH), jnp.int32),
        pltpu.VMEM((NCH, CH), jnp.int32),
        [pltpu.VMEM((CH, HIDDEN), jnp.float32)] * 5,
        pltpu.VMEM_SHARED((PADN, HIDDEN), jnp.float32),
        [pltpu.SemaphoreType.DMA] * 5,
        [pltpu.SemaphoreType.DMA] * 5,
    ],
)
def _sc_propagate(g_hbm, srcs_hbm, dsts_hbm, out_hbm,
                  src_v, dst_v, rows, acc, gsems, ssems):
    c = lax.axis_index("c")
    s = lax.axis_index("s")

    def zrow(i, carry):
        col = (i % 4) * 16
        rows[0][i // 4, pl.ds(col, 16)] = jnp.zeros((16,), jnp.float32)
        return carry

    lax.fori_loop(0, CH * 4, zrow, 0)
    for k in range(5):
        n = CH if k < 4 else RPT - 4 * CH
        pltpu.sync_copy(rows[0].at[pl.ds(0, n)],
                        acc.at[pl.ds(s * RPT + k * CH, n)])
    plsc.subcore_barrier()

    def chunk(i, carry):
        j = i * 5
        g = [pltpu.async_copy(g_hbm.at[src_v.at[j + k]], rows[k], gsems[k])
             for k in range(5)]
        sc = []
        for k in range(5):
            g[k].wait()
            sc.append(pltpu.async_copy(rows[k], acc.at[dst_v.at[j + k]],
                                       ssems[k], add=True))
        for k in range(5):
            sc[k].wait()
        return carry

    wid = c * NS + s

    @pl.when(wid < NTILES - 1)
    def _():
        pltpu.sync_copy(srcs_hbm.at[pl.ds(wid * NCH, NCH)], src_v)
        pltpu.sync_copy(dsts_hbm.at[pl.ds(wid * NCH, NCH)], dst_v)

    @pl.when(wid == NTILES - 1)
    def _():
        pltpu.sync_copy(srcs_hbm.at[pl.ds((NTILES - 1) * NCH, TAIL_NCH)],
                        src_v.at[pl.ds(0, TAIL_NCH)])
        pltpu.sync_copy(dsts_hbm.at[pl.ds((NTILES - 1) * NCH, TAIL_NCH)],
                        dst_v.at[pl.ds(0, TAIL_NCH)])

    lax.fori_loop(0, jnp.where(wid == NTILES - 1, TAIL_NCH // 5, NCH // 5),
                  chunk, 0)
    plsc.subcore_barrier()
    pltpu.sync_copy(acc.at[pl.ds(s * RPT, RPT)],
                    out_hbm.at[c, pl.ds(s * RPT, RPT)])


def _tc1_body(x_ref, w1_ref, d0_ref, d1_ref, dinv_ref, g1_ref):
    deg = d0_ref[...] + d1_ref[...] + 1.0
    dinv = lax.rsqrt(deg)
    dinv_ref[...] = dinv
    h = jnp.dot(x_ref[...], w1_ref[...], preferred_element_type=jnp.float32)
    g1_ref[...] = h * dinv


def _tc2_body(p0_ref, p1_ref, g1_ref, dinv_ref, b1_ref, w2_ref, g2_ref):
    dinv = dinv_ref[...]
    pre = dinv * (p0_ref[...] + p1_ref[...] + g1_ref[...]) + b1_ref[...]
    out1 = jnp.maximum(pre, 0.0)
    g2_ref[...] = jnp.dot(out1, w2_ref[...],
                          preferred_element_type=jnp.float32) * dinv


def _tc3_body(q0_ref, q1_ref, g2_ref, dinv_ref, b2_ref, batch_ref,
              wc_ref, bc_ref, out_ref, acc_ref):
    i = pl.program_id(0)

    @pl.when(i == 0)
    def _():
        acc_ref[...] = jnp.zeros_like(acc_ref)

    dinv = dinv_ref[...]
    pre = dinv * (q0_ref[...] + q1_ref[...] + g2_ref[...]) + b2_ref[...]
    h = jnp.maximum(pre, 0.0)
    hx = jnp.concatenate([h, jnp.ones((BLK, 1), jnp.float32)], axis=1)
    docs = lax.broadcasted_iota(jnp.int32, (N_DOCS, 1), 0).astype(jnp.float32)
    onehot_t = (docs == batch_ref[0]).astype(jnp.float32)
    acc_ref[...] += jnp.dot(onehot_t, hx,
                            preferred_element_type=jnp.float32)

    @pl.when(i == NBLK - 1)
    def _():
        sums = acc_ref[:, :HIDDEN]
        counts = acc_ref[:, HIDDEN:HIDDEN + 1]
        pooled = sums / jnp.maximum(counts, 1.0)
        logits = jnp.dot(pooled, wc_ref[...],
                         preferred_element_type=jnp.float32) + bc_ref[...]
        m = jnp.max(logits, axis=1, keepdims=True)
        lse = jnp.log(jnp.sum(jnp.exp(logits - m), axis=1, keepdims=True)) + m
        out_ref[...] = logits - lse


_tc1 = pl.pallas_call(
    _tc1_body,
    grid=(NBLK,),
    in_specs=[
        pl.BlockSpec((BLK, VOCAB), lambda i: (i, 0)),
        pl.BlockSpec((VOCAB, HIDDEN), lambda i: (0, 0)),
        pl.BlockSpec((BLK, 1), lambda i: (i, 0)),
        pl.BlockSpec((BLK, 1), lambda i: (i, 0)),
    ],
    out_specs=[
        pl.BlockSpec((BLK, 1), lambda i: (i, 0)),
        pl.BlockSpec((BLK, HIDDEN), lambda i: (i, 0)),
    ],
    out_shape=[
        jax.ShapeDtypeStruct((N_NODES, 1), jnp.float32),
        jax.ShapeDtypeStruct((N_NODES, HIDDEN), jnp.float32),
    ],
)

_tc2 = pl.pallas_call(
    _tc2_body,
    grid=(NBLK,),
    in_specs=[
        pl.BlockSpec((BLK, HIDDEN), lambda i: (i, 0)),
        pl.BlockSpec((BLK, HIDDEN), lambda i: (i, 0)),
        pl.BlockSpec((BLK, HIDDEN), lambda i: (i, 0)),
        pl.BlockSpec((BLK, 1), lambda i: (i, 0)),
        pl.BlockSpec((1, HIDDEN), lambda i: (0, 0)),
        pl.BlockSpec((HIDDEN, HIDDEN), lambda i: (0, 0)),
    ],
    out_specs=pl.BlockSpec((BLK, HIDDEN), lambda i: (i, 0)),
    out_shape=jax.ShapeDtypeStruct((N_NODES, HIDDEN), jnp.float32),
)

_tc3 = pl.pallas_call(
    _tc3_body,
    grid=(NBLK,),
    in_specs=[
        pl.BlockSpec((BLK, HIDDEN), lambda i: (i, 0)),
        pl.BlockSpec((BLK, HIDDEN), lambda i: (i, 0)),
        pl.BlockSpec((BLK, HIDDEN), lambda i: (i, 0)),
        pl.BlockSpec((BLK, 1), lambda i: (i, 0)),
        pl.BlockSpec((1, HIDDEN), lambda i: (0, 0)),
        pl.BlockSpec((1, 1, BLK), lambda i: (i, 0, 0)),
        pl.BlockSpec((HIDDEN, N_CLASSES), lambda i: (0, 0)),
        pl.BlockSpec((1, N_CLASSES), lambda i: (0, 0)),
    ],
    out_specs=pl.BlockSpec((N_DOCS, N_CLASSES), lambda i: (0, 0)),
    out_shape=jax.ShapeDtypeStruct((N_DOCS, N_CLASSES), jnp.float32),
    scratch_shapes=[pltpu.VMEM((N_DOCS, HIDDEN + 1), jnp.float32)],
)


def kernel(x, edge_index, batch, W1, b1, W2, b2, Wc, bc):
    src_p = edge_index[0].astype(jnp.int32).reshape(CHUNKS, CH)
    dst_p = edge_index[1].astype(jnp.int32).reshape(CHUNKS, CH)

    zerosd = jnp.zeros((DEGR, 16), jnp.float32)
    ids = jnp.arange(DEGR, dtype=jnp.int32).reshape(5, CH)

    degw = _sc_degree(dst_p, zerosd, ids)
    d0 = degw[0].reshape(DEGR * 16)[:N_NODES, None]
    d1 = degw[1].reshape(DEGR * 16)[:N_NODES, None]

    dinv, g1 = _tc1(x, W1, d0, d1)

    P = _sc_propagate(g1, src_p, dst_p)
    g2 = _tc2(P[0, :N_NODES], P[1, :N_NODES], g1, dinv,
              b1.reshape(1, HIDDEN), W2)

    Q = _sc_propagate(g2, src_p, dst_p)
    out = _tc3(Q[0, :N_NODES], Q[1, :N_NODES], g2, dinv,
               b2.reshape(1, HIDDEN),
               batch.astype(jnp.float32).reshape(NBLK, 1, BLK),
               Wc, bc.reshape(1, N_CLASSES))
    return out
